# baseline (speedup 1.0000x reference)
import functools

import jax
import jax.numpy as jnp
from jax.experimental import pallas as pl
from jax.experimental.pallas import tpu as pltpu

NEG_SLOPE = 0.5


def _leaky(x):
    return jnp.where(x >= 0, x, NEG_SLOPE * x)


def _pick_block(n):
    for b in (4000, 2000, 1000, 500, 250, 125, 100, 50, 25, 8):
        if n % b == 0:
            return b
    return n


def _hgnn_body(adj_ref, emb_ref, out_ref, adj_sc, lat_sc, *, blk, nblk):
    p = pl.program_id(0)
    i = pl.program_id(1)

    @pl.when(p == 0)
    def _phase0():
        a = adj_ref[...].astype(jnp.bfloat16)
        adj_sc[pl.ds(i * blk, blk), :] = a
        part = jax.lax.dot_general(
            a,
            emb_ref[...].astype(jnp.bfloat16),
            (((0,), (0,)), ((), ())),
            preferred_element_type=jnp.float32,
        )

        @pl.when(i == 0)
        def _():
            lat_sc[...] = part

        @pl.when(i > 0)
        def _():
            lat_sc[...] += part

    @pl.when(p == 1)
    def _phase1():
        lat = _leaky(lat_sc[...]).astype(jnp.bfloat16)
        r = jax.lax.dot_general(
            adj_sc[pl.ds(i * blk, blk), :],
            lat,
            (((1,), (0,)), ((), ())),
            preferred_element_type=jnp.float32,
        )
        out_ref[...] = _leaky(r)


@jax.jit
def kernel(adj, embeds):
    n, h = adj.shape
    d = embeds.shape[1]
    blk = _pick_block(n)
    nblk = n // blk

    body = functools.partial(_hgnn_body, blk=blk, nblk=nblk)
    return pl.pallas_call(
        body,
        grid=(2, nblk),
        in_specs=[
            pl.BlockSpec((blk, h), lambda p, i: (i * (1 - p), 0)),
            pl.BlockSpec((blk, d), lambda p, i: (i * (1 - p), 0)),
        ],
        out_specs=pl.BlockSpec((blk, d), lambda p, i: (i * p, 0)),
        out_shape=jax.ShapeDtypeStruct((n, d), jnp.float32),
        scratch_shapes=[
            pltpu.VMEM((n, h), jnp.bfloat16),
            pltpu.VMEM((h, d), jnp.float32),
        ],
    )(adj, embeds)


# speedup vs baseline: 1.0061x; 1.0061x over previous
import functools

import jax
import jax.numpy as jnp
from jax.experimental import pallas as pl
from jax.experimental.pallas import tpu as pltpu

NEG_SLOPE = 0.5


def _leaky(x):
    return jnp.where(x >= 0, x, NEG_SLOPE * x)


def _pick_block(n):
    for b in (4000, 2000, 1000, 500, 250, 125, 100, 50, 25, 8):
        if n % b == 0:
            return b
    return n


def _hgnn_body(adj_ref, emb_ref, out_ref, adj_sc, latT_sc, lat_sc, *, blk, nblk):
    p = pl.program_id(0)
    i = pl.program_id(1)

    @pl.when(p == 0)
    def _phase0():
        a = adj_ref[...].astype(jnp.bfloat16)
        adj_sc[pl.ds(i * blk, blk), :] = a
        et = jnp.swapaxes(emb_ref[...].astype(jnp.bfloat16), 0, 1)
        part = jax.lax.dot_general(
            et,
            a,
            (((1,), (0,)), ((), ())),
            preferred_element_type=jnp.float32,
        )

        @pl.when(i == 0)
        def _():
            latT_sc[...] = part

        @pl.when(i > 0)
        def _():
            latT_sc[...] += part

    @pl.when(p == 1)
    def _phase1():
        @pl.when(i == 0)
        def _():
            lat_sc[...] = jnp.swapaxes(_leaky(latT_sc[...]), 0, 1).astype(
                jnp.bfloat16
            )

        r = jax.lax.dot_general(
            adj_sc[pl.ds(i * blk, blk), :],
            lat_sc[...],
            (((1,), (0,)), ((), ())),
            preferred_element_type=jnp.float32,
        )
        out_ref[...] = _leaky(r)


@jax.jit
def kernel(adj, embeds):
    n, h = adj.shape
    d = embeds.shape[1]
    blk = _pick_block(n)
    nblk = n // blk

    body = functools.partial(_hgnn_body, blk=blk, nblk=nblk)
    return pl.pallas_call(
        body,
        grid=(2, nblk),
        in_specs=[
            pl.BlockSpec((blk, h), lambda p, i: (i * (1 - p), 0)),
            pl.BlockSpec((blk, d), lambda p, i: (i * (1 - p), 0)),
        ],
        out_specs=pl.BlockSpec((blk, d), lambda p, i: (i * p, 0)),
        out_shape=jax.ShapeDtypeStruct((n, d), jnp.float32),
        scratch_shapes=[
            pltpu.VMEM((n, h), jnp.bfloat16),
            pltpu.VMEM((d, h), jnp.float32),
            pltpu.VMEM((h, d), jnp.bfloat16),
        ],
    )(adj, embeds)


# speedup vs baseline: 1.1080x; 1.1013x over previous
import functools

import jax
import jax.numpy as jnp
from jax.experimental import pallas as pl
from jax.experimental.pallas import tpu as pltpu

NEG_SLOPE = 0.5
SPLIT = 8


def _leaky(x):
    return jnp.where(x >= 0, x, NEG_SLOPE * x)


def _pick_sub(n):
    for sub in (1250, 1000, 625, 500, 250, 125, 100, 25, 5, 1):
        if n % (SPLIT * sub) == 0:
            return sub
    return 0


def _hgnn_body(adj_ref, embT_ref, outT_ref, adj_sc, latT_sc, lat_sc, *, sub, h):
    p = pl.program_id(0)
    i = pl.program_id(1)

    @pl.when(p == 0)
    def _phase0():
        def sub_dot(k, acc):
            ak = adj_ref[k].astype(jnp.bfloat16)
            adj_sc[pl.ds(i * SPLIT + k, 1), :, :] = ak[None]
            return acc + jax.lax.dot_general(
                embT_ref[:, k, :].astype(jnp.bfloat16),
                ak,
                (((1,), (0,)), ((), ())),
                preferred_element_type=jnp.float32,
            )

        part = sub_dot(0, jnp.zeros_like(latT_sc))
        for k in range(1, SPLIT):
            part = sub_dot(k, part)

        @pl.when(i == 0)
        def _():
            latT_sc[...] = part

        @pl.when(i > 0)
        def _():
            latT_sc[...] += part

    @pl.when(p == 1)
    def _phase1():
        @pl.when(i == 0)
        def _():
            lat_sc[...] = jnp.swapaxes(_leaky(latT_sc[...]), 0, 1).astype(
                jnp.bfloat16
            )

        for k in range(SPLIT):
            ak = adj_sc[pl.ds(i * SPLIT + k, 1), :, :].reshape(sub, h)
            r = jax.lax.dot_general(
                ak,
                lat_sc[...],
                (((1,), (0,)), ((), ())),
                preferred_element_type=jnp.float32,
            )
            outT_ref[:, k, :] = jnp.swapaxes(_leaky(r), 0, 1)


@jax.jit
def kernel(adj, embeds):
    n, h = adj.shape
    d = embeds.shape[1]
    sub = _pick_sub(n)
    nblk = n // (SPLIT * sub)
    nsub = n // sub

    adj3 = adj.reshape(nsub, sub, h)
    embT = embeds.T.reshape(d, nsub, sub)
    body = functools.partial(_hgnn_body, sub=sub, h=h)
    retT = pl.pallas_call(
        body,
        grid=(2, nblk),
        in_specs=[
            pl.BlockSpec((SPLIT, sub, h), lambda p, i: (i * (1 - p), 0, 0)),
            pl.BlockSpec((d, SPLIT, sub), lambda p, i: (0, i * (1 - p), 0)),
        ],
        out_specs=pl.BlockSpec((d, SPLIT, sub), lambda p, i: (0, i * p, 0)),
        out_shape=jax.ShapeDtypeStruct((d, nsub, sub), jnp.float32),
        scratch_shapes=[
            pltpu.VMEM((nsub, sub, h), jnp.bfloat16),
            pltpu.VMEM((d, h), jnp.float32),
            pltpu.VMEM((h, d), jnp.bfloat16),
        ],
    )(adj3, embT)
    return retT.reshape(d, n).T


# speedup vs baseline: 2.2427x; 2.0241x over previous
import functools

import jax
import jax.numpy as jnp
from jax.experimental import pallas as pl
from jax.experimental.pallas import tpu as pltpu

NEG_SLOPE = 0.5
W = 3840


def _leaky(x):
    return jnp.where(x >= 0, x, NEG_SLOPE * x)


def _dot(x, y):
    return jax.lax.dot_general(
        x, y, (((1,), (0,)), ((), ())), preferred_element_type=jnp.float32
    )


def _hgnn_body(adj_ref, embT_ref, outT_ref, adj_sc, latT_sc, lat_sc,
               *, nblk, rem, h, d):
    p = pl.program_id(0)
    i = pl.program_id(1)

    @pl.when(p == 0)
    def _phase0():
        ab = adj_ref[...].astype(jnp.bfloat16)
        e = embT_ref[...].astype(jnp.bfloat16)

        @pl.when(i < nblk - 1)
        def _full():
            adj_sc[pl.ds(i * W, W), :] = ab
            part = _dot(e, ab)

            @pl.when(i == 0)
            def _():
                latT_sc[...] = part

            @pl.when(i > 0)
            def _():
                latT_sc[...] += part

        @pl.when(i == nblk - 1)
        def _partial():
            rowmask = jax.lax.broadcasted_iota(jnp.int32, (W, h), 0) < rem
            ab2 = jnp.where(rowmask, ab, jnp.bfloat16(0))
            adj_sc[pl.ds(i * W, W), :] = ab2
            lanemask = jax.lax.broadcasted_iota(jnp.int32, (d, W), 1) < rem
            e2 = jnp.where(lanemask, e, jnp.bfloat16(0))
            latT_sc[...] += _dot(e2, ab2)

    @pl.when(p == 1)
    def _phase1():
        @pl.when(i == 0)
        def _():
            lat_sc[...] = jnp.swapaxes(_leaky(latT_sc[...]), 0, 1).astype(
                jnp.bfloat16
            )

        r = _dot(adj_sc[pl.ds(i * W, W), :], lat_sc[...])
        outT_ref[...] = jnp.swapaxes(_leaky(r), 0, 1)


@jax.jit
def kernel(adj, embeds):
    n, h = adj.shape
    d = embeds.shape[1]
    nblk = -(-n // W)
    rem = n - (nblk - 1) * W

    embT = embeds.T
    body = functools.partial(_hgnn_body, nblk=nblk, rem=rem, h=h, d=d)
    retT = pl.pallas_call(
        body,
        grid=(2, nblk),
        in_specs=[
            pl.BlockSpec((W, h), lambda p, i: (i * (1 - p), 0)),
            pl.BlockSpec((d, W), lambda p, i: (0, i * (1 - p))),
        ],
        out_specs=pl.BlockSpec((d, W), lambda p, i: (0, i * p)),
        out_shape=jax.ShapeDtypeStruct((d, n), jnp.float32),
        scratch_shapes=[
            pltpu.VMEM((nblk * W, h), jnp.bfloat16),
            pltpu.VMEM((d, h), jnp.float32),
            pltpu.VMEM((h, d), jnp.bfloat16),
        ],
    )(adj, embT)
    return retT.T


# speedup vs baseline: 3.1061x; 1.3850x over previous
import functools

import jax
import jax.numpy as jnp
from jax.experimental import pallas as pl
from jax.experimental.pallas import tpu as pltpu

NEG_SLOPE = 0.5
W = 7680


def _leaky(x):
    return jnp.where(x >= 0, x, NEG_SLOPE * x)


def _dot(x, y):
    return jax.lax.dot_general(
        x, y, (((1,), (0,)), ((), ())), preferred_element_type=jnp.float32
    )


def _hgnn_body(adj_ref, embT_ref, outT_ref, adj_sc, latT_sc, lat_sc,
               *, nblk, rem, h, d):
    p = pl.program_id(0)
    i = pl.program_id(1)

    @pl.when(p == 0)
    def _phase0():
        ab = adj_ref[...].astype(jnp.bfloat16)
        e = embT_ref[...].astype(jnp.bfloat16)

        @pl.when(i < nblk - 1)
        def _full():
            adj_sc[pl.ds(i * W, W), :] = ab
            part = _dot(e, ab)

            @pl.when(i == 0)
            def _():
                latT_sc[...] = part

            @pl.when(i > 0)
            def _():
                latT_sc[...] += part

        @pl.when(i == nblk - 1)
        def _partial():
            rowmask = jax.lax.broadcasted_iota(jnp.int32, (W, h), 0) < rem
            ab2 = jnp.where(rowmask, ab, jnp.bfloat16(0))
            adj_sc[pl.ds(i * W, W), :] = ab2
            lanemask = jax.lax.broadcasted_iota(jnp.int32, (d, W), 1) < rem
            e2 = jnp.where(lanemask, e, jnp.bfloat16(0))
            latT_sc[...] += _dot(e2, ab2)

    @pl.when(p == 1)
    def _phase1():
        @pl.when(i == 0)
        def _():
            lat_sc[...] = jnp.swapaxes(_leaky(latT_sc[...]), 0, 1).astype(
                jnp.bfloat16
            )

        r = _dot(adj_sc[pl.ds(i * W, W), :], lat_sc[...]).astype(jnp.bfloat16)
        outT_ref[...] = jnp.swapaxes(_leaky(r), 0, 1).astype(jnp.float32)


@jax.jit
def kernel(adj, embeds):
    n, h = adj.shape
    d = embeds.shape[1]
    nblk = -(-n // W)
    rem = n - (nblk - 1) * W

    embT = embeds.T
    body = functools.partial(_hgnn_body, nblk=nblk, rem=rem, h=h, d=d)
    retT = pl.pallas_call(
        body,
        grid=(2, nblk),
        in_specs=[
            pl.BlockSpec((W, h), lambda p, i: (i * (1 - p), 0)),
            pl.BlockSpec((d, W), lambda p, i: (0, i * (1 - p))),
        ],
        out_specs=pl.BlockSpec((d, W), lambda p, i: (0, i * p)),
        out_shape=jax.ShapeDtypeStruct((d, n), jnp.float32),
        scratch_shapes=[
            pltpu.VMEM((nblk * W, h), jnp.bfloat16),
            pltpu.VMEM((d, h), jnp.float32),
            pltpu.VMEM((h, d), jnp.bfloat16),
        ],
    )(adj, embT)
    return retT.T


# speedup vs baseline: 3.2932x; 1.0602x over previous
import functools

import jax
import jax.numpy as jnp
from jax.experimental import pallas as pl
from jax.experimental.pallas import tpu as pltpu

NEG_SLOPE = 0.5
W = 9984


def _leaky(x):
    return jnp.where(x >= 0, x, NEG_SLOPE * x)


def _dot(x, y):
    return jax.lax.dot_general(
        x, y, (((1,), (0,)), ((), ())), preferred_element_type=jnp.float32
    )


def _hgnn_body(adj_ref, embT_ref, outT_ref, adj_sc, latT_sc, lat_sc,
               *, nblk, rem, h, d):
    p = pl.program_id(0)
    i = pl.program_id(1)

    @pl.when(p == 0)
    def _phase0():
        ab = adj_ref[...].astype(jnp.bfloat16)
        e = embT_ref[...].astype(jnp.bfloat16)

        @pl.when(i < nblk - 1)
        def _full():
            adj_sc[pl.ds(i * W, W), :] = ab
            part = _dot(e, ab)

            @pl.when(i == 0)
            def _():
                latT_sc[...] = part

            @pl.when(i > 0)
            def _():
                latT_sc[...] += part

        @pl.when(i == nblk - 1)
        def _partial():
            rowmask = jax.lax.broadcasted_iota(jnp.int32, (W, h), 0) < rem
            ab2 = jnp.where(rowmask, ab, jnp.bfloat16(0))
            adj_sc[pl.ds(i * W, W), :] = ab2
            lanemask = jax.lax.broadcasted_iota(jnp.int32, (d, W), 1) < rem
            e2 = jnp.where(lanemask, e, jnp.bfloat16(0))
            latT_sc[...] += _dot(e2, ab2)

    @pl.when(p == 1)
    def _phase1():
        @pl.when(i == 0)
        def _():
            lat_sc[...] = jnp.swapaxes(_leaky(latT_sc[...]), 0, 1).astype(
                jnp.bfloat16
            )

        r = _dot(adj_sc[pl.ds(i * W, W), :], lat_sc[...]).astype(jnp.bfloat16)
        outT_ref[...] = jnp.swapaxes(_leaky(r), 0, 1).astype(jnp.float32)


@jax.jit
def kernel(adj, embeds):
    n, h = adj.shape
    d = embeds.shape[1]
    nblk = -(-n // W)
    rem = n - (nblk - 1) * W

    embT = embeds.T
    body = functools.partial(_hgnn_body, nblk=nblk, rem=rem, h=h, d=d)
    retT = pl.pallas_call(
        body,
        grid=(2, nblk),
        in_specs=[
            pl.BlockSpec((W, h), lambda p, i: (i * (1 - p), 0)),
            pl.BlockSpec((d, W), lambda p, i: (0, i * (1 - p))),
        ],
        out_specs=pl.BlockSpec((d, W), lambda p, i: (0, i * p)),
        out_shape=jax.ShapeDtypeStruct((d, n), jnp.float32),
        scratch_shapes=[
            pltpu.VMEM((nblk * W, h), jnp.bfloat16),
            pltpu.VMEM((d, h), jnp.float32),
            pltpu.VMEM((h, d), jnp.bfloat16),
        ],
    )(adj, embT)
    return retT.T


# speedup vs baseline: 3.3962x; 1.0313x over previous
import functools

import jax
import jax.numpy as jnp
from jax.experimental import pallas as pl
from jax.experimental.pallas import tpu as pltpu

NEG_SLOPE = 0.5
W = 9984


def _leaky(x):
    return jnp.where(x >= 0, x, NEG_SLOPE * x)


def _dot(x, y):
    return jax.lax.dot_general(
        x, y, (((1,), (0,)), ((), ())), preferred_element_type=jnp.float32
    )


def _hgnn_body(adj_ref, embT_ref, outT_ref, adj_sc, latT_sc, lat_sc,
               *, nblk, rem, h, d):
    p = pl.program_id(0)
    i = pl.program_id(1)

    @pl.when(p == 0)
    def _phase0():
        ab = adj_ref[...].astype(jnp.bfloat16)
        e = embT_ref[...].astype(jnp.bfloat16)

        @pl.when(i < nblk - 1)
        def _full():
            adj_sc[pl.ds(i * W, W), :] = ab
            part = _dot(e, ab)

            @pl.when(i == 0)
            def _():
                latT_sc[...] = part

            @pl.when(i > 0)
            def _():
                latT_sc[...] += part

        @pl.when(i == nblk - 1)
        def _partial():
            rowmask = jax.lax.broadcasted_iota(jnp.int32, (W, h), 0) < rem
            ab2 = jnp.where(rowmask, ab, jnp.bfloat16(0))
            adj_sc[pl.ds(i * W, W), :] = ab2
            lanemask = jax.lax.broadcasted_iota(jnp.int32, (d, W), 1) < rem
            e2 = jnp.where(lanemask, e, jnp.bfloat16(0))
            latT_sc[...] += _dot(e2, ab2)

    @pl.when(p == 1)
    def _phase1():
        @pl.when(i == 0)
        def _():
            lat_sc[...] = jnp.swapaxes(_leaky(latT_sc[...]), 0, 1).astype(
                jnp.bfloat16
            )

        j = nblk - 1 - i
        off = 0
        while off < W:
            sw = min(2560, W - off)
            r = _dot(
                adj_sc[pl.ds(j * W + off, sw), :], lat_sc[...]
            ).astype(jnp.bfloat16)
            outT_ref[:, off:off + sw] = jnp.swapaxes(_leaky(r), 0, 1).astype(
                jnp.float32
            )
            off += sw


@jax.jit
def kernel(adj, embeds):
    n, h = adj.shape
    d = embeds.shape[1]
    nblk = -(-n // W)
    rem = n - (nblk - 1) * W

    embT = embeds.T
    body = functools.partial(_hgnn_body, nblk=nblk, rem=rem, h=h, d=d)
    retT = pl.pallas_call(
        body,
        grid=(2, nblk),
        in_specs=[
            pl.BlockSpec((W, h),
                         lambda p, i: (i * (1 - p) + (nblk - 1) * p, 0)),
            pl.BlockSpec((d, W),
                         lambda p, i: (0, i * (1 - p) + (nblk - 1) * p)),
        ],
        out_specs=pl.BlockSpec((d, W), lambda p, i: (0, nblk - 1 - i * p)),
        out_shape=jax.ShapeDtypeStruct((d, n), jnp.float32),
        scratch_shapes=[
            pltpu.VMEM((nblk * W, h), jnp.bfloat16),
            pltpu.VMEM((d, h), jnp.float32),
            pltpu.VMEM((h, d), jnp.bfloat16),
        ],
    )(adj, embT)
    return retT.T


# speedup vs baseline: 3.6913x; 1.0869x over previous
import functools

import jax
import jax.numpy as jnp
from jax.experimental import pallas as pl
from jax.experimental.pallas import tpu as pltpu

NEG_SLOPE = 0.5
W = 9984


def _leaky(x):
    return jnp.where(x >= 0, x, NEG_SLOPE * x)


def _dot(x, y):
    return jax.lax.dot_general(
        x, y, (((1,), (0,)), ((), ())), preferred_element_type=jnp.float32
    )


def _hgnn_body(adj_ref, embT_ref, outT_ref, adj_sc, latT_sc, lat_sc,
               *, nblk, rem, h, d):
    p = pl.program_id(0)
    i = pl.program_id(1)

    @pl.when(p == 0)
    def _phase0():
        ab = adj_ref[...].astype(jnp.bfloat16)
        e = embT_ref[...].astype(jnp.bfloat16)

        @pl.when(i < nblk - 1)
        def _full():
            adj_sc[:, pl.ds(i * W, W)] = jnp.swapaxes(ab, 0, 1)
            part = _dot(e, ab)

            @pl.when(i == 0)
            def _():
                latT_sc[...] = part

            @pl.when(i > 0)
            def _():
                latT_sc[...] += part

        @pl.when(i == nblk - 1)
        def _partial():
            rowmask = jax.lax.broadcasted_iota(jnp.int32, (W, h), 0) < rem
            ab2 = jnp.where(rowmask, ab, jnp.bfloat16(0))
            adj_sc[:, pl.ds(i * W, W)] = jnp.swapaxes(ab2, 0, 1)
            lanemask = jax.lax.broadcasted_iota(jnp.int32, (d, W), 1) < rem
            e2 = jnp.where(lanemask, e, jnp.bfloat16(0))
            latT_sc[...] += _dot(e2, ab2)

    @pl.when(p == 1)
    def _phase1():
        @pl.when(i == 0)
        def _():
            lat_sc[...] = _leaky(latT_sc[...]).astype(jnp.bfloat16)

        j = nblk - 1 - i
        outT_ref[...] = _leaky(_dot(lat_sc[...], adj_sc[:, pl.ds(j * W, W)]))


@jax.jit
def kernel(adj, embeds):
    n, h = adj.shape
    d = embeds.shape[1]
    nblk = -(-n // W)
    rem = n - (nblk - 1) * W

    embT = embeds.T
    body = functools.partial(_hgnn_body, nblk=nblk, rem=rem, h=h, d=d)
    retT = pl.pallas_call(
        body,
        grid=(2, nblk),
        in_specs=[
            pl.BlockSpec((W, h),
                         lambda p, i: (i * (1 - p) + (nblk - 1) * p, 0)),
            pl.BlockSpec((d, W),
                         lambda p, i: (0, i * (1 - p) + (nblk - 1) * p)),
        ],
        out_specs=pl.BlockSpec((d, W), lambda p, i: (0, nblk - 1 - i * p)),
        out_shape=jax.ShapeDtypeStruct((d, n), jnp.float32),
        scratch_shapes=[
            pltpu.VMEM((h, nblk * W), jnp.bfloat16),
            pltpu.VMEM((d, h), jnp.float32),
            pltpu.VMEM((d, h), jnp.bfloat16),
        ],
    )(adj, embT)
    return retT.T
